# Initial kernel scaffold; baseline (speedup 1.0000x reference)
import jax
import jax.numpy as jnp
from jax.experimental import pallas as pl


def kernel(x, edge_index, edge_attr, node_W, node_b, edge_W, edge_b, ee_W, ee_b, pre_W, pre_b, post_W, post_b, lin_W, lin_b, em1_W, em1_b, em2_W, em2_b, bn_w, bn_b):
    raise NotImplementedError("write your pallas kernel here")


# speedup vs baseline: 1.4536x; 1.4536x over previous
import functools
import math

import jax
import jax.numpy as jnp
from jax import lax
from jax.experimental import pallas as pl
from jax.experimental.pallas import tpu as pltpu
from jax.experimental.pallas import tpu_sc as plsc

N = 10000
E = 320000
H = 128
AVG_LOG = math.log(33.0)

BE = 1600
BN = 1000
BS = 2000


_SC_NC = 2
_SC_NS = 16
_NW = _SC_NC * _SC_NS


@functools.lru_cache(maxsize=None)
def _sc_gather(Mrows, Q):
    qpw = Q // _NW
    CH = 400
    assert qpw % CH == 0 and Q % _NW == 0
    mesh = plsc.VectorSubcoreMesh(core_axis_name="c", subcore_axis_name="s")

    @functools.partial(
        pl.kernel,
        mesh=mesh,
        out_type=jax.ShapeDtypeStruct((Q, H), jnp.float32),
        scratch_types=[
            pltpu.VMEM((CH,), jnp.int32),
            pltpu.VMEM((CH, H), jnp.float32),
            pltpu.SemaphoreType.DMA,
        ],
    )
    def gk(table_hbm, idx_hbm, out_hbm, idx_v, rows_v, sem):
        wid = lax.axis_index("s") * _SC_NC + lax.axis_index("c")
        w0 = pl.multiple_of(wid * qpw, 8)

        def body(c, carry):
            base = pl.multiple_of(w0 + c * CH, 8)
            pltpu.sync_copy(idx_hbm.at[pl.ds(base, CH)], idx_v)
            pltpu.async_copy(table_hbm.at[idx_v], rows_v, sem).wait()
            pltpu.sync_copy(rows_v, out_hbm.at[pl.ds(base, CH)])
            return carry

        lax.fori_loop(0, qpw // CH, body, 0)

    return gk


def _gather(table, idx):
    return _sc_gather(table.shape[0], idx.shape[0])(table, idx)


def _dot(a, b):
    return jnp.dot(a, b, preferred_element_type=jnp.float32)


def _prep_kernel(eeW_ref, eeb_ref, preW_ref, preb_ref, W3p_ref, bf_ref):
    for i in range(2):
        W3 = preW_ref[i][2 * H:, :]
        W3p_ref[i] = _dot(eeW_ref[i], W3)
        bf_ref[i] = preb_ref[i] + _dot(eeb_ref[i], W3)


def _prep(ee_W, ee_b, pre_W, pre_b):
    return pl.pallas_call(
        _prep_kernel,
        out_shape=(
            jax.ShapeDtypeStruct((2, H, H), jnp.float32),
            jax.ShapeDtypeStruct((2, 1, H), jnp.float32),
        ),
    )(ee_W, ee_b.reshape(2, 1, H), pre_W, pre_b.reshape(2, 1, H))


def _encN_kernel(x_ref, W_ref, b_ref, o_ref):
    o_ref[...] = _dot(x_ref[...], W_ref[...]) + b_ref[...]


def _enc_nodes(x, W, b):
    return pl.pallas_call(
        _encN_kernel,
        grid=(N // BN,),
        in_specs=[
            pl.BlockSpec((BN, H), lambda i: (i, 0)),
            pl.BlockSpec((H, H), lambda i: (0, 0)),
            pl.BlockSpec((1, H), lambda i: (0, 0)),
        ],
        out_specs=pl.BlockSpec((BN, H), lambda i: (i, 0)),
        out_shape=jax.ShapeDtypeStruct((N, H), jnp.float32),
    )(x, W, b.reshape(1, H))


def _enc_edges(ea, W, b):
    D = ea.shape[1]
    return pl.pallas_call(
        _encN_kernel,
        grid=(E // BE,),
        in_specs=[
            pl.BlockSpec((BE, D), lambda i: (i, 0)),
            pl.BlockSpec((D, H), lambda i: (0, 0)),
            pl.BlockSpec((1, H), lambda i: (0, 0)),
        ],
        out_specs=pl.BlockSpec((BE, H), lambda i: (i, 0)),
        out_shape=jax.ShapeDtypeStruct((E, H), jnp.float32),
    )(ea, W, b.reshape(1, H))


def _tables_kernel(x_ref, W_ref, o_ref):
    o_ref[0] = _dot(x_ref[...], W_ref[0])


def _tables(x, Wstack):
    S = Wstack.shape[0]
    out = pl.pallas_call(
        _tables_kernel,
        grid=(S, N // BN),
        in_specs=[
            pl.BlockSpec((BN, H), lambda s, k: (k, 0)),
            pl.BlockSpec((1, H, H), lambda s, k: (s, 0, 0)),
        ],
        out_specs=pl.BlockSpec((1, BN, H), lambda s, k: (s, k, 0)),
        out_shape=jax.ShapeDtypeStruct((S, N, H), jnp.float32),
    )(x, Wstack)
    return out.reshape(S * N, H)


def _mker_kernel(gd_ref, gs_ref, ea_ref, W3p_ref, bf_ref, m_ref):
    m_ref[...] = (gd_ref[...] + gs_ref[...]
                  + _dot(ea_ref[...], W3p_ref[...]) + bf_ref[...])


def _m_edges(g, ea, W3p, bf):
    KB = E // BE
    return pl.pallas_call(
        _mker_kernel,
        grid=(KB,),
        in_specs=[
            pl.BlockSpec((BE, H), lambda i: (i, 0)),
            pl.BlockSpec((BE, H), lambda i: (i + KB, 0)),
            pl.BlockSpec((BE, H), lambda i: (i, 0)),
            pl.BlockSpec((H, H), lambda i: (0, 0)),
            pl.BlockSpec((1, H), lambda i: (0, 0)),
        ],
        out_specs=pl.BlockSpec((BE, H), lambda i: (i, 0)),
        out_shape=jax.ShapeDtypeStruct((E, H), jnp.float32),
    )(g, g, ea, W3p, bf)


def _scatter_kernel(m_ref, idx_ref, s_ref, s2_ref, mn_ref, mx_ref, cnt_ref):
    @pl.when(pl.program_id(0) == 0)
    def _init():
        zero = jnp.zeros((N, H), jnp.float32)
        s_ref[...] = zero
        s2_ref[...] = zero
        cnt_ref[...] = zero
        mn_ref[...] = jnp.full((N, H), jnp.inf, jnp.float32)
        mx_ref[...] = jnp.full((N, H), -jnp.inf, jnp.float32)

    def body(j, carry):
        d = idx_ref[0, 0, j]
        row = m_ref[pl.ds(j, 1), :]
        s_ref[pl.ds(d, 1), :] = s_ref[pl.ds(d, 1), :] + row
        s2_ref[pl.ds(d, 1), :] = s2_ref[pl.ds(d, 1), :] + row * row
        mn_ref[pl.ds(d, 1), :] = jnp.minimum(mn_ref[pl.ds(d, 1), :], row)
        mx_ref[pl.ds(d, 1), :] = jnp.maximum(mx_ref[pl.ds(d, 1), :], row)
        cnt_ref[pl.ds(d, 1), :] = cnt_ref[pl.ds(d, 1), :] + 1.0
        return carry

    lax.fori_loop(0, BS, body, 0)


def _scatter(m, dst3):
    outs = [jax.ShapeDtypeStruct((N, H), jnp.float32)] * 5
    full = pl.BlockSpec((N, H), lambda i: (0, 0))
    return pl.pallas_call(
        _scatter_kernel,
        grid=(E // BS,),
        in_specs=[
            pl.BlockSpec((BS, H), lambda i: (i, 0)),
            pl.BlockSpec((1, 1, BS), lambda i: (i, 0, 0), memory_space=pltpu.SMEM),
        ],
        out_specs=[full] * 5,
        out_shape=outs,
    )(m, dst3)


def _nodeA_kernel(s_ref, s2_ref, mn_ref, mx_ref, cnt_ref, x_ref, PW_ref,
                  pb_ref, lin_ref, lb_ref, out_ref, cs_ref, cq_ref):
    cnt = cnt_ref[...]
    deg = jnp.maximum(cnt, 1.0)
    has = cnt > 0.0
    mean = s_ref[...] / deg
    std = jnp.sqrt(jax.nn.relu(s2_ref[...] / deg - mean * mean) + 1e-5)
    mn = jnp.where(has, mn_ref[...], 0.0)
    mx = jnp.where(has, mx_ref[...], 0.0)
    logd = jnp.log(deg + 1.0)
    amp = logd * (1.0 / AVG_LOG)
    att = AVG_LOG / logd
    A = (mean, mn, mx, std)
    t0 = _dot(x_ref[...], PW_ref[0])
    t1 = sum(_dot(A[k], PW_ref[1 + k]) for k in range(4))
    t2 = sum(_dot(A[k], PW_ref[5 + k]) for k in range(4))
    t3 = sum(_dot(A[k], PW_ref[9 + k]) for k in range(4))
    out = t0 + t1 + amp * t2 + att * t3 + pb_ref[...]
    out = _dot(out, lin_ref[...]) + lb_ref[...]
    out_ref[...] = out

    @pl.when(pl.program_id(0) == 0)
    def _init():
        cs_ref[...] = jnp.zeros((1, H), jnp.float32)
        cq_ref[...] = jnp.zeros((1, H), jnp.float32)

    cs_ref[...] = cs_ref[...] + jnp.sum(out, axis=0, keepdims=True)
    cq_ref[...] = cq_ref[...] + jnp.sum(out * out, axis=0, keepdims=True)


def _nodeA(s, s2, mn, mx, cnt, x, PW, pb, lin, lb):
    blk = pl.BlockSpec((BN, H), lambda i: (i, 0))
    one = pl.BlockSpec((1, H), lambda i: (0, 0))
    return pl.pallas_call(
        _nodeA_kernel,
        grid=(N // BN,),
        in_specs=[blk, blk, blk, blk, blk, blk,
                  pl.BlockSpec((13, H, H), lambda i: (0, 0, 0)),
                  one,
                  pl.BlockSpec((H, H), lambda i: (0, 0)),
                  one],
        out_specs=[blk, one, one],
        out_shape=[
            jax.ShapeDtypeStruct((N, H), jnp.float32),
            jax.ShapeDtypeStruct((1, H), jnp.float32),
            jax.ShapeDtypeStruct((1, H), jnp.float32),
        ],
    )(s, s2, mn, mx, cnt, x, PW, pb.reshape(1, H), lin, lb.reshape(1, H))


def _nodeB_kernel(out_ref, cs_ref, cq_ref, x_ref, bw_ref, bb_ref, xn_ref):
    mu = cs_ref[...] * (1.0 / N)
    var = cq_ref[...] * (1.0 / N) - mu * mu
    inv = jax.lax.rsqrt(var + 1e-5)
    bn = (out_ref[...] - mu) * inv * bw_ref[...] + bb_ref[...]
    xn_ref[...] = (x_ref[...] + jax.nn.relu(bn)) * 0.5


def _nodeB(out, cs, cq, x, bw, bb):
    blk = pl.BlockSpec((BN, H), lambda i: (i, 0))
    one = pl.BlockSpec((1, H), lambda i: (0, 0))
    return pl.pallas_call(
        _nodeB_kernel,
        grid=(N // BN,),
        in_specs=[blk, one, one, blk, one, one],
        out_specs=blk,
        out_shape=jax.ShapeDtypeStruct((N, H), jnp.float32),
    )(out, cs, cq, x, bw.reshape(1, H), bb.reshape(1, H))


def _em_kernel(ga_ref, gb_ref, ea_ref, B3_ref, b1_ref, W2_ref, b2_ref, o_ref):
    hid = (ga_ref[...] + gb_ref[...]
           + _dot(ea_ref[...], B3_ref[...]) + b1_ref[...])
    em = _dot(jax.nn.relu(hid), W2_ref[...]) + b2_ref[...]
    o_ref[...] = ea_ref[...] + em * 0.5


def _em_edges(g, off_a, off_b, ea, B3, b1, W2, b2):
    KB = E // BE
    return pl.pallas_call(
        _em_kernel,
        grid=(KB,),
        in_specs=[
            pl.BlockSpec((BE, H), lambda i, o=off_a: (i + o * KB, 0)),
            pl.BlockSpec((BE, H), lambda i, o=off_b: (i + o * KB, 0)),
            pl.BlockSpec((BE, H), lambda i: (i, 0)),
            pl.BlockSpec((H, H), lambda i: (0, 0)),
            pl.BlockSpec((1, H), lambda i: (0, 0)),
            pl.BlockSpec((H, H), lambda i: (0, 0)),
            pl.BlockSpec((1, H), lambda i: (0, 0)),
        ],
        out_specs=pl.BlockSpec((BE, H), lambda i: (i, 0)),
        out_shape=jax.ShapeDtypeStruct((E, H), jnp.float32),
    )(g, g, ea, B3, b1.reshape(1, H), W2, b2.reshape(1, H))


def kernel(x, edge_index, edge_attr, node_W, node_b, edge_W, edge_b, ee_W,
           ee_b, pre_W, pre_b, post_W, post_b, lin_W, lin_b, em1_W, em1_b,
           em2_W, em2_b, bn_w, bn_b):
    src = edge_index[0]
    dst = edge_index[1]
    dst3 = dst.reshape(E // BS, 1, BS)

    W3p, bf = _prep(ee_W, ee_b, pre_W, pre_b)

    xc = _enc_nodes(x, node_W, node_b)
    ea = _enc_edges(edge_attr, edge_W, edge_b)

    W1 = [pre_W[i][:H] for i in range(2)]
    W2 = [pre_W[i][H:2 * H] for i in range(2)]
    B1 = [em1_W[i][:H] for i in range(2)]
    B2 = [em1_W[i][H:2 * H] for i in range(2)]
    B3 = [em1_W[i][2 * H:] for i in range(2)]
    PW = [post_W[i].reshape(13, H, H) for i in range(2)]

    idx_ds = jnp.concatenate([dst, src + N])
    idx_emn = jnp.concatenate([src, dst + N, dst + 2 * N, src + 3 * N])
    idx_em = jnp.concatenate([src, dst + N])

    cnt = None
    for i in range(2):
        if i == 0:
            T = _tables(xc, jnp.stack([W1[0], W2[0]]))
            g = _gather(T, idx_ds)
            m = _m_edges(g, ea, W3p[i], bf[i])
        s, s2, mn, mx, cnt_i = _scatter(m, dst3)
        if cnt is None:
            cnt = cnt_i
        out, cs, cq = _nodeA(s, s2, mn, mx, cnt, xc, PW[i], post_b[i],
                             lin_W[i], lin_b[i])
        xc = _nodeB(out, cs, cq, xc, bn_w[i], bn_b[i])
        if i == 0:
            T = _tables(xc, jnp.stack([B1[0], B2[0], W1[1], W2[1]]))
            g = _gather(T, idx_emn)
            ea = _em_edges(g, 0, 1, ea, B3[0], em1_b[0], em2_W[0], em2_b[0])
            m = _m_edges(g[2 * E:], ea, W3p[1], bf[1])
        else:
            T = _tables(xc, jnp.stack([B1[1], B2[1]]))
            g = _gather(T, idx_em)
            ea = _em_edges(g, 0, 1, ea, B3[1], em1_b[1], em2_W[1], em2_b[1])
    return xc, ea
